# baseline (speedup 1.0000x reference)
import functools

import jax
import jax.numpy as jnp
from jax import lax
from jax.experimental import pallas as pl
from jax.experimental.pallas import tpu as pltpu
from jax.experimental.pallas import tpu_sc as plsc

_BD = 20.0
_BM = 0.1
_NS = 1000
_TAB_PAD = 1024

_NC = 2
_NSUB = 16
_LANES = 16
_NW = _NC * _NSUB

_B = 512
_D = 3 * 128 * 128
_ROWS = 8
_TW = 128


def _coeff_table():
    ts = jnp.linspace(0.0, 1.0, _NS, dtype=jnp.float32)
    betas = (_BM + (_BD - _BM) * ts) / _NS
    alphas = 1.0 - betas
    ac = jnp.cumprod(alphas, axis=0)
    sqrt_recip = jnp.sqrt(1.0 / ac)
    sqrt_recipm1 = jnp.sqrt(1.0 / ac - 1.0)
    tab = jnp.zeros((_NS, _TW), jnp.float32)
    tab = tab.at[:, 0].set(sqrt_recip)
    tab = tab.at[:, 1].set(sqrt_recipm1)
    return tab


def _sc_gather_body(tab_hbm, idx_hbm, out_hbm, idx_v, rows_v, sem):
    wid = lax.axis_index("s") * _NC + lax.axis_index("c")
    base = wid * _LANES
    pltpu.sync_copy(idx_hbm.at[pl.ds(base, _LANES)], idx_v)
    pltpu.async_copy(tab_hbm.at[idx_v], rows_v, sem).wait()
    pltpu.sync_copy(rows_v, out_hbm.at[pl.ds(base, _LANES)])


@functools.lru_cache(maxsize=1)
def _sc_gather():
    return pl.kernel(
        _sc_gather_body,
        out_type=jax.ShapeDtypeStruct((_B, _TW), jnp.float32),
        mesh=plsc.VectorSubcoreMesh(core_axis_name="c", subcore_axis_name="s"),
        scratch_types=[
            pltpu.VMEM((_LANES,), jnp.int32),
            pltpu.VMEM((_LANES, _TW), jnp.float32),
            pltpu.SemaphoreType.DMA,
        ],
    )


def _tc_fma_body(c_ref, x_ref, n_ref, o_ref):
    a = c_ref[:, 0:1]
    b = c_ref[:, 1:2]
    o_ref[...] = a * x_ref[...] - b * n_ref[...]


def _tc_fma(coeffs, x2, n2):
    grid = (_B // _ROWS,)
    coeff_spec = pl.BlockSpec((_ROWS, _TW), lambda r: (r, 0))
    row_spec = pl.BlockSpec((_ROWS, _D), lambda r: (r, 0))
    return pl.pallas_call(
        _tc_fma_body,
        grid=grid,
        in_specs=[coeff_spec, row_spec, row_spec],
        out_specs=row_spec,
        out_shape=jax.ShapeDtypeStruct((_B, _D), jnp.float32),
    )(coeffs, x2, n2)


def kernel(x_i, noise, i):
    tab = _coeff_table()
    coeffs = _sc_gather()(tab, i.astype(jnp.int32))
    x2 = x_i.reshape(_B, _D)
    n2 = noise.reshape(_B, _D)
    out = _tc_fma(coeffs, x2, n2)
    return out.reshape(x_i.shape)


# speedup vs baseline: 3.2170x; 3.2170x over previous
import functools

import jax
import jax.numpy as jnp
from jax import lax
from jax.experimental import pallas as pl
from jax.experimental.pallas import tpu as pltpu
from jax.experimental.pallas import tpu_sc as plsc

_BD = 20.0
_BM = 0.1
_NS = 1000
_TAB_PAD = 1024

_NC = 2
_NSUB = 16
_LANES = 16
_NW = _NC * _NSUB

_B = 512
_D = 3 * 128 * 128
_ROWS = 8
_TW = 128


def _coeff_table():
    ts = jnp.linspace(0.0, 1.0, _NS, dtype=jnp.float32)
    betas = (_BM + (_BD - _BM) * ts) / _NS
    alphas = 1.0 - betas
    ac = jnp.cumprod(alphas, axis=0)
    sqrt_recip = jnp.sqrt(1.0 / ac)
    sqrt_recipm1 = jnp.sqrt(1.0 / ac - 1.0)
    tab = jnp.zeros((_NS, _TW), jnp.float32)
    tab = tab.at[:, 0].set(sqrt_recip)
    tab = tab.at[:, 1].set(sqrt_recipm1)
    return tab


def _sc_gather_body(tab_hbm, idx_hbm, out_hbm, idx_v, rows_v, sem):
    wid = lax.axis_index("s") * _NC + lax.axis_index("c")
    base = wid * _LANES
    pltpu.sync_copy(idx_hbm.at[pl.ds(base, _LANES)], idx_v)
    pltpu.async_copy(tab_hbm.at[idx_v], rows_v, sem).wait()
    pltpu.sync_copy(rows_v, out_hbm.at[pl.ds(base, _LANES)])


@functools.lru_cache(maxsize=1)
def _sc_gather():
    return pl.kernel(
        _sc_gather_body,
        out_type=jax.ShapeDtypeStruct((_B, _TW), jnp.float32),
        mesh=plsc.VectorSubcoreMesh(core_axis_name="c", subcore_axis_name="s"),
        scratch_types=[
            pltpu.VMEM((_LANES,), jnp.int32),
            pltpu.VMEM((_LANES, _TW), jnp.float32),
            pltpu.SemaphoreType.DMA,
        ],
    )


def _tc_fma_body(c_ref, x_ref, n_ref, o_ref):
    a = c_ref[:, 0:1].reshape(_ROWS, 1, 1, 1)
    b = c_ref[:, 1:2].reshape(_ROWS, 1, 1, 1)
    o_ref[...] = a * x_ref[...] - b * n_ref[...]


def _tc_fma(coeffs, x4, n4):
    grid = (_B // _ROWS,)
    coeff_spec = pl.BlockSpec((_ROWS, _TW), lambda r: (r, 0))
    row_spec = pl.BlockSpec((_ROWS, 3, 128, 128), lambda r: (r, 0, 0, 0))
    return pl.pallas_call(
        _tc_fma_body,
        grid=grid,
        in_specs=[coeff_spec, row_spec, row_spec],
        out_specs=row_spec,
        out_shape=jax.ShapeDtypeStruct((_B, 3, 128, 128), jnp.float32),
    )(coeffs, x4, n4)


def kernel(x_i, noise, i):
    tab = _coeff_table()
    coeffs = _sc_gather()(tab, i.astype(jnp.int32))
    return _tc_fma(coeffs, x_i, noise)


# speedup vs baseline: 3.4038x; 1.0581x over previous
import functools

import jax
import jax.numpy as jnp
from jax import lax
from jax.experimental import pallas as pl
from jax.experimental.pallas import tpu as pltpu
from jax.experimental.pallas import tpu_sc as plsc

_BD = 20.0
_BM = 0.1
_NS = 1000
_TAB_PAD = 1024

_NC = 2
_NSUB = 16
_LANES = 16
_NW = _NC * _NSUB

_B = 512
_D = 3 * 128 * 128
_ROWS = 16
_TW = 128


def _coeff_table():
    ts = jnp.linspace(0.0, 1.0, _NS, dtype=jnp.float32)
    betas = (_BM + (_BD - _BM) * ts) / _NS
    alphas = 1.0 - betas
    ac = jnp.cumprod(alphas, axis=0)
    sqrt_recip = jnp.sqrt(1.0 / ac)
    sqrt_recipm1 = jnp.sqrt(1.0 / ac - 1.0)
    tab = jnp.zeros((_NS, _TW), jnp.float32)
    tab = tab.at[:, 0].set(sqrt_recip)
    tab = tab.at[:, 1].set(sqrt_recipm1)
    return tab


def _sc_gather_body(tab_hbm, idx_hbm, out_hbm, idx_v, rows_v, sem):
    wid = lax.axis_index("s") * _NC + lax.axis_index("c")
    base = wid * _LANES
    pltpu.sync_copy(idx_hbm.at[pl.ds(base, _LANES)], idx_v)
    pltpu.async_copy(tab_hbm.at[idx_v], rows_v, sem).wait()
    pltpu.sync_copy(rows_v, out_hbm.at[pl.ds(base, _LANES)])


@functools.lru_cache(maxsize=1)
def _sc_gather():
    return pl.kernel(
        _sc_gather_body,
        out_type=jax.ShapeDtypeStruct((_B, _TW), jnp.float32),
        mesh=plsc.VectorSubcoreMesh(core_axis_name="c", subcore_axis_name="s"),
        scratch_types=[
            pltpu.VMEM((_LANES,), jnp.int32),
            pltpu.VMEM((_LANES, _TW), jnp.float32),
            pltpu.SemaphoreType.DMA,
        ],
    )


def _tc_fma_body(c_ref, x_ref, n_ref, o_ref):
    a = c_ref[:, 0:1].reshape(_ROWS, 1, 1, 1)
    b = c_ref[:, 1:2].reshape(_ROWS, 1, 1, 1)
    o_ref[...] = a * x_ref[...] - b * n_ref[...]


def _tc_fma(coeffs, x4, n4):
    grid = (_B // _ROWS,)
    coeff_spec = pl.BlockSpec((_ROWS, _TW), lambda r: (r, 0))
    row_spec = pl.BlockSpec((_ROWS, 3, 128, 128), lambda r: (r, 0, 0, 0))
    return pl.pallas_call(
        _tc_fma_body,
        grid=grid,
        in_specs=[coeff_spec, row_spec, row_spec],
        out_specs=row_spec,
        out_shape=jax.ShapeDtypeStruct((_B, 3, 128, 128), jnp.float32),
    )(coeffs, x4, n4)


def kernel(x_i, noise, i):
    tab = _coeff_table()
    coeffs = _sc_gather()(tab, i.astype(jnp.int32))
    return _tc_fma(coeffs, x_i, noise)


# speedup vs baseline: 3.4569x; 1.0156x over previous
import functools

import jax
import jax.numpy as jnp
from jax import lax
from jax.experimental import pallas as pl
from jax.experimental.pallas import tpu as pltpu
from jax.experimental.pallas import tpu_sc as plsc

_BD = 20.0
_BM = 0.1
_NS = 1000
_TAB_PAD = 1024

_NC = 2
_NSUB = 16
_LANES = 16
_NW = _NC * _NSUB

_B = 512
_D = 3 * 128 * 128
_ROWS = 32
_TW = 128


def _coeff_table():
    ts = jnp.linspace(0.0, 1.0, _NS, dtype=jnp.float32)
    betas = (_BM + (_BD - _BM) * ts) / _NS
    alphas = 1.0 - betas
    ac = jnp.cumprod(alphas, axis=0)
    sqrt_recip = jnp.sqrt(1.0 / ac)
    sqrt_recipm1 = jnp.sqrt(1.0 / ac - 1.0)
    tab = jnp.zeros((_NS, _TW), jnp.float32)
    tab = tab.at[:, 0].set(sqrt_recip)
    tab = tab.at[:, 1].set(sqrt_recipm1)
    return tab


def _sc_gather_body(tab_hbm, idx_hbm, out_hbm, idx_v, rows_v, sem):
    wid = lax.axis_index("s") * _NC + lax.axis_index("c")
    base = wid * _LANES
    pltpu.sync_copy(idx_hbm.at[pl.ds(base, _LANES)], idx_v)
    pltpu.async_copy(tab_hbm.at[idx_v], rows_v, sem).wait()
    pltpu.sync_copy(rows_v, out_hbm.at[pl.ds(base, _LANES)])


@functools.lru_cache(maxsize=1)
def _sc_gather():
    return pl.kernel(
        _sc_gather_body,
        out_type=jax.ShapeDtypeStruct((_B, _TW), jnp.float32),
        mesh=plsc.VectorSubcoreMesh(core_axis_name="c", subcore_axis_name="s"),
        scratch_types=[
            pltpu.VMEM((_LANES,), jnp.int32),
            pltpu.VMEM((_LANES, _TW), jnp.float32),
            pltpu.SemaphoreType.DMA,
        ],
    )


def _tc_fma_body(c_ref, x_ref, n_ref, o_ref):
    a = c_ref[:, 0:1].reshape(_ROWS, 1, 1, 1)
    b = c_ref[:, 1:2].reshape(_ROWS, 1, 1, 1)
    o_ref[...] = a * x_ref[...] - b * n_ref[...]


def _tc_fma(coeffs, x4, n4):
    grid = (_B // _ROWS,)
    coeff_spec = pl.BlockSpec((_ROWS, _TW), lambda r: (r, 0))
    row_spec = pl.BlockSpec((_ROWS, 3, 128, 128), lambda r: (r, 0, 0, 0))
    return pl.pallas_call(
        _tc_fma_body,
        grid=grid,
        in_specs=[coeff_spec, row_spec, row_spec],
        out_specs=row_spec,
        out_shape=jax.ShapeDtypeStruct((_B, 3, 128, 128), jnp.float32),
    )(coeffs, x4, n4)


def kernel(x_i, noise, i):
    tab = _coeff_table()
    coeffs = _sc_gather()(tab, i.astype(jnp.int32))
    return _tc_fma(coeffs, x_i, noise)
